# Initial kernel scaffold; baseline (speedup 1.0000x reference)
import jax
import jax.numpy as jnp
from jax.experimental import pallas as pl


def kernel(pos_xyz, cel_mat, sft_cel, spc, adj_ij, sft_idx, rc):
    raise NotImplementedError("write your pallas kernel here")


# speedup vs baseline: 66.4663x; 66.4663x over previous
import functools

import jax
import jax.numpy as jnp
from jax import lax
from jax.experimental import pallas as pl
from jax.experimental.pallas import tpu as pltpu
from jax.experimental.pallas import tpu_sc as plsc

_N = 50000
_E = 3200000
_NW = 32
_EW = _E // _NW
_C = 2000
_NCHUNK = _EW // _C
_VPC = _C // 16

_mesh = plsc.VectorSubcoreMesh(core_axis_name="c", subcore_axis_name="s")
_params = pltpu.CompilerParams(needs_layout_passes=False)


def _worker_id():
    return lax.axis_index("s") * 2 + lax.axis_index("c")


def _pass_a_body(adj_hbm, sft_hbm, tabx_hbm, taby_hbm, sftx_hbm, sfty_hbm,
                 part_hbm, tabx, taby, sftx, sfty, bi, bj, bs, bp):
    base_w = _worker_id() * _EW
    pltpu.sync_copy(tabx_hbm, tabx)
    pltpu.sync_copy(taby_hbm, taby)
    pltpu.sync_copy(sftx_hbm, sftx)
    pltpu.sync_copy(sfty_hbm, sfty)

    def chunk(k, carry):
        b = base_w + k * _C
        pltpu.sync_copy(adj_hbm.at[pl.ds(b, _C)], bi)
        pltpu.sync_copy(adj_hbm.at[pl.ds(_E + b, _C)], bj)
        pltpu.sync_copy(sft_hbm.at[pl.ds(b, _C)], bs)

        def vec(v, c):
            o = v * 16
            iv = bi[pl.ds(o, 16)]
            jv = bj[pl.ds(o, 16)]
            sv = bs[pl.ds(o, 16)]
            xj = plsc.load_gather(tabx, [jv])
            xi = plsc.load_gather(tabx, [iv])
            yj = plsc.load_gather(taby, [jv])
            yi = plsc.load_gather(taby, [iv])
            tx = plsc.load_gather(sftx, [sv])
            ty = plsc.load_gather(sfty, [sv])
            dx = xj - xi + tx
            dy = yj - yi + ty
            bp[pl.ds(o, 16)] = dx * dx + dy * dy
            return c

        lax.fori_loop(0, _VPC, vec, 0)
        pltpu.sync_copy(bp, part_hbm.at[pl.ds(b, _C)])
        return carry

    lax.fori_loop(0, _NCHUNK, chunk, 0)


def _pass_b_body(adj_hbm, sft_hbm, part_hbm, tabz_hbm, sftz_hbm, rc2_hbm,
                 adj_out_hbm, sod_hbm,
                 tabz, sftz, rc2v, bi, bj, bs, bp, bsod, b0, b1):
    base_w = _worker_id() * _EW
    pltpu.sync_copy(tabz_hbm, tabz)
    pltpu.sync_copy(sftz_hbm, sftz)
    pltpu.sync_copy(rc2_hbm, rc2v)
    rc2 = rc2v[...]
    neg1 = jnp.full((16,), -1, jnp.int32)

    def chunk(k, carry):
        b = base_w + k * _C
        pltpu.sync_copy(adj_hbm.at[pl.ds(b, _C)], bi)
        pltpu.sync_copy(adj_hbm.at[pl.ds(_E + b, _C)], bj)
        pltpu.sync_copy(sft_hbm.at[pl.ds(b, _C)], bs)
        pltpu.sync_copy(part_hbm.at[pl.ds(b, _C)], bp)

        def vec(v, c):
            o = v * 16
            iv = bi[pl.ds(o, 16)]
            jv = bj[pl.ds(o, 16)]
            sv = bs[pl.ds(o, 16)]
            pv = bp[pl.ds(o, 16)]
            zj = plsc.load_gather(tabz, [jv])
            zi = plsc.load_gather(tabz, [iv])
            tz = plsc.load_gather(sftz, [sv])
            dz = zj - zi + tz
            sod = pv + dz * dz
            m = sod <= rc2
            bsod[pl.ds(o, 16)] = sod
            b0[pl.ds(o, 16)] = jnp.where(m, iv, neg1)
            b1[pl.ds(o, 16)] = jnp.where(m, jv, neg1)
            return c

        lax.fori_loop(0, _VPC, vec, 0)
        pltpu.sync_copy(bsod, sod_hbm.at[pl.ds(b, _C)])
        pltpu.sync_copy(b0, adj_out_hbm.at[pl.ds(b, _C)])
        pltpu.sync_copy(b1, adj_out_hbm.at[pl.ds(_E + b, _C)])
        return carry

    lax.fori_loop(0, _NCHUNK, chunk, 0)


_pass_a = functools.partial(
    pl.kernel,
    out_type=jax.ShapeDtypeStruct((_E,), jnp.float32),
    mesh=_mesh,
    compiler_params=_params,
    scratch_types=[
        pltpu.VMEM((_N,), jnp.float32),
        pltpu.VMEM((_N,), jnp.float32),
        pltpu.VMEM((32,), jnp.float32),
        pltpu.VMEM((32,), jnp.float32),
        pltpu.VMEM((_C,), jnp.int32),
        pltpu.VMEM((_C,), jnp.int32),
        pltpu.VMEM((_C,), jnp.int32),
        pltpu.VMEM((_C,), jnp.float32),
    ],
)(_pass_a_body)

_pass_b = functools.partial(
    pl.kernel,
    out_type=(jax.ShapeDtypeStruct((2 * _E,), jnp.int32),
              jax.ShapeDtypeStruct((_E,), jnp.float32)),
    mesh=_mesh,
    compiler_params=_params,
    scratch_types=[
        pltpu.VMEM((_N,), jnp.float32),
        pltpu.VMEM((32,), jnp.float32),
        pltpu.VMEM((16,), jnp.float32),
        pltpu.VMEM((_C,), jnp.int32),
        pltpu.VMEM((_C,), jnp.int32),
        pltpu.VMEM((_C,), jnp.int32),
        pltpu.VMEM((_C,), jnp.float32),
        pltpu.VMEM((_C,), jnp.float32),
        pltpu.VMEM((_C,), jnp.int32),
        pltpu.VMEM((_C,), jnp.int32),
    ],
)(_pass_b_body)


def kernel(pos_xyz, cel_mat, sft_cel, spc, adj_ij, sft_idx, rc):
    cel = cel_mat[0]
    pos = pos_xyz[0] + spc[0].astype(jnp.float32) @ cel
    sftm = sft_cel @ cel
    sft_pad = jnp.pad(sftm, ((0, 5), (0, 0)))
    rc_f = jnp.asarray(rc, jnp.float32)
    rc2v = jnp.full((16,), rc_f * rc_f, jnp.float32)

    tabx = jnp.copy(pos[:, 0])
    taby = jnp.copy(pos[:, 1])
    tabz = jnp.copy(pos[:, 2])
    sftx = jnp.copy(sft_pad[:, 0])
    sfty = jnp.copy(sft_pad[:, 1])
    sftz = jnp.copy(sft_pad[:, 2])

    adj_flat = adj_ij.reshape(2 * _E)
    part = _pass_a(adj_flat, sft_idx, tabx, taby, sftx, sfty)
    adj_out_flat, sod = _pass_b(adj_flat, sft_idx, part, tabz, sftz, rc2v)
    return adj_out_flat.reshape(2, _E), sod


# speedup vs baseline: 98.2574x; 1.4783x over previous
import functools

import jax
import jax.numpy as jnp
from jax import lax
from jax.experimental import pallas as pl
from jax.experimental.pallas import tpu as pltpu
from jax.experimental.pallas import tpu_sc as plsc

_N = 50000
_E = 3200000
_NW = 32
_EW = _E // _NW
_C = 2000
_NCHUNK = _EW // _C
_NPAIR = _NCHUNK // 2

_mesh = plsc.VectorSubcoreMesh(core_axis_name="c", subcore_axis_name="s")
_params = pltpu.CompilerParams(needs_layout_passes=False)


def _worker_id():
    return lax.axis_index("s") * 2 + lax.axis_index("c")


def _pass_a_body(adj_hbm, sft_hbm, tabx_hbm, taby_hbm, sftx_hbm, sfty_hbm,
                 part_hbm,
                 tabx, taby, sftx, sfty,
                 bi0, bj0, bs0, bp0, bi1, bj1, bs1, bp1,
                 sin0, sin1, sout0, sout1):
    base_w = _worker_id() * _EW
    pltpu.sync_copy(tabx_hbm, tabx)
    pltpu.sync_copy(taby_hbm, taby)
    pltpu.sync_copy(sftx_hbm, sftx)
    pltpu.sync_copy(sfty_hbm, sfty)

    buf = ((bi0, bj0, bs0, bp0, sin0, sout0),
           (bi1, bj1, bs1, bp1, sin1, sout1))

    def issue_in(s, b):
        bi, bj, bs, _, sin, _ = buf[s]
        pltpu.async_copy(adj_hbm.at[pl.ds(b, _C)], bi, sin)
        pltpu.async_copy(adj_hbm.at[pl.ds(_E + b, _C)], bj, sin)
        pltpu.async_copy(sft_hbm.at[pl.ds(b, _C)], bs, sin)

    def wait_in(s):
        bi, bj, bs, _, sin, _ = buf[s]
        pltpu.make_async_copy(adj_hbm.at[pl.ds(0, _C)], bi, sin).wait()
        pltpu.make_async_copy(adj_hbm.at[pl.ds(0, _C)], bj, sin).wait()
        pltpu.make_async_copy(sft_hbm.at[pl.ds(0, _C)], bs, sin).wait()

    def compute(s):
        bi, bj, bs, bp, _, _ = buf[s]

        @plsc.parallel_loop(0, _C, step=16, unroll=5)
        def _(o):
            iv = bi[pl.ds(o, 16)]
            jv = bj[pl.ds(o, 16)]
            sv = bs[pl.ds(o, 16)]
            xj = plsc.load_gather(tabx, [jv])
            xi = plsc.load_gather(tabx, [iv])
            yj = plsc.load_gather(taby, [jv])
            yi = plsc.load_gather(taby, [iv])
            tx = plsc.load_gather(sftx, [sv])
            ty = plsc.load_gather(sfty, [sv])
            dx = xj - xi + tx
            dy = yj - yi + ty
            bp[pl.ds(o, 16)] = dx * dx + dy * dy

    def issue_out(s, b):
        _, _, _, bp, _, sout = buf[s]
        pltpu.async_copy(bp, part_hbm.at[pl.ds(b, _C)], sout)

    def wait_out(s):
        _, _, _, bp, _, sout = buf[s]
        pltpu.make_async_copy(bp, part_hbm.at[pl.ds(0, _C)], sout).wait()

    issue_in(0, base_w)

    def pair(t, carry):
        b0 = base_w + (2 * t) * _C
        b2 = jnp.minimum(b0 + 2 * _C, _E - _C)
        issue_in(1, b0 + _C)
        wait_in(0)

        @pl.when(t > 0)
        def _():
            wait_out(0)

        compute(0)
        issue_out(0, b0)
        issue_in(0, b2)
        wait_in(1)

        @pl.when(t > 0)
        def _():
            wait_out(1)

        compute(1)
        issue_out(1, b0 + _C)
        return carry

    lax.fori_loop(0, _NPAIR, pair, 0)
    wait_in(0)
    wait_out(0)
    wait_out(1)


def _pass_b_body(adj_hbm, sft_hbm, part_hbm, tabz_hbm, sftz_hbm, rc2_hbm,
                 adj_out_hbm, sod_hbm,
                 tabz, sftz, rc2v,
                 bi0, bj0, bs0, bp0, bd0, b00, b10,
                 bi1, bj1, bs1, bp1, bd1, b01, b11,
                 sin0, sin1, sout0, sout1):
    base_w = _worker_id() * _EW
    pltpu.sync_copy(tabz_hbm, tabz)
    pltpu.sync_copy(sftz_hbm, sftz)
    pltpu.sync_copy(rc2_hbm, rc2v)
    rc2 = rc2v[...]
    neg1 = jnp.full((16,), -1, jnp.int32)

    buf = ((bi0, bj0, bs0, bp0, bd0, b00, b10, sin0, sout0),
           (bi1, bj1, bs1, bp1, bd1, b01, b11, sin1, sout1))

    def issue_in(s, b):
        bi, bj, bs, bp, _, _, _, sin, _ = buf[s]
        pltpu.async_copy(adj_hbm.at[pl.ds(b, _C)], bi, sin)
        pltpu.async_copy(adj_hbm.at[pl.ds(_E + b, _C)], bj, sin)
        pltpu.async_copy(sft_hbm.at[pl.ds(b, _C)], bs, sin)
        pltpu.async_copy(part_hbm.at[pl.ds(b, _C)], bp, sin)

    def wait_in(s):
        bi, bj, bs, bp, _, _, _, sin, _ = buf[s]
        pltpu.make_async_copy(adj_hbm.at[pl.ds(0, _C)], bi, sin).wait()
        pltpu.make_async_copy(adj_hbm.at[pl.ds(0, _C)], bj, sin).wait()
        pltpu.make_async_copy(sft_hbm.at[pl.ds(0, _C)], bs, sin).wait()
        pltpu.make_async_copy(part_hbm.at[pl.ds(0, _C)], bp, sin).wait()

    def compute(s):
        bi, bj, bs, bp, bd, b0, b1, _, _ = buf[s]

        @plsc.parallel_loop(0, _C, step=16, unroll=5)
        def _(o):
            iv = bi[pl.ds(o, 16)]
            jv = bj[pl.ds(o, 16)]
            sv = bs[pl.ds(o, 16)]
            pv = bp[pl.ds(o, 16)]
            zj = plsc.load_gather(tabz, [jv])
            zi = plsc.load_gather(tabz, [iv])
            tz = plsc.load_gather(sftz, [sv])
            dz = zj - zi + tz
            sod = pv + dz * dz
            m = sod <= rc2
            bd[pl.ds(o, 16)] = sod
            b0[pl.ds(o, 16)] = jnp.where(m, iv, neg1)
            b1[pl.ds(o, 16)] = jnp.where(m, jv, neg1)

    def issue_out(s, b):
        _, _, _, _, bd, b0, b1, _, sout = buf[s]
        pltpu.async_copy(bd, sod_hbm.at[pl.ds(b, _C)], sout)
        pltpu.async_copy(b0, adj_out_hbm.at[pl.ds(b, _C)], sout)
        pltpu.async_copy(b1, adj_out_hbm.at[pl.ds(_E + b, _C)], sout)

    def wait_out(s):
        _, _, _, _, bd, b0, b1, _, sout = buf[s]
        pltpu.make_async_copy(bd, sod_hbm.at[pl.ds(0, _C)], sout).wait()
        pltpu.make_async_copy(b0, adj_out_hbm.at[pl.ds(0, _C)], sout).wait()
        pltpu.make_async_copy(b1, adj_out_hbm.at[pl.ds(0, _C)], sout).wait()

    issue_in(0, base_w)

    def pair(t, carry):
        b0 = base_w + (2 * t) * _C
        b2 = jnp.minimum(b0 + 2 * _C, _E - _C)
        issue_in(1, b0 + _C)
        wait_in(0)

        @pl.when(t > 0)
        def _():
            wait_out(0)

        compute(0)
        issue_out(0, b0)
        issue_in(0, b2)
        wait_in(1)

        @pl.when(t > 0)
        def _():
            wait_out(1)

        compute(1)
        issue_out(1, b0 + _C)
        return carry

    lax.fori_loop(0, _NPAIR, pair, 0)
    wait_in(0)
    wait_out(0)
    wait_out(1)


_pass_a = functools.partial(
    pl.kernel,
    out_type=jax.ShapeDtypeStruct((_E,), jnp.float32),
    mesh=_mesh,
    compiler_params=_params,
    scratch_types=(
        [pltpu.VMEM((_N,), jnp.float32)] * 2
        + [pltpu.VMEM((32,), jnp.float32)] * 2
        + [pltpu.VMEM((_C,), jnp.int32)] * 3 + [pltpu.VMEM((_C,), jnp.float32)]
        + [pltpu.VMEM((_C,), jnp.int32)] * 3 + [pltpu.VMEM((_C,), jnp.float32)]
        + [pltpu.SemaphoreType.DMA] * 4
    ),
)(_pass_a_body)

_pass_b = functools.partial(
    pl.kernel,
    out_type=(jax.ShapeDtypeStruct((2 * _E,), jnp.int32),
              jax.ShapeDtypeStruct((_E,), jnp.float32)),
    mesh=_mesh,
    compiler_params=_params,
    scratch_types=(
        [pltpu.VMEM((_N,), jnp.float32),
         pltpu.VMEM((32,), jnp.float32),
         pltpu.VMEM((16,), jnp.float32)]
        + [pltpu.VMEM((_C,), jnp.int32)] * 3
        + [pltpu.VMEM((_C,), jnp.float32)] * 2
        + [pltpu.VMEM((_C,), jnp.int32)] * 2
        + [pltpu.VMEM((_C,), jnp.int32)] * 3
        + [pltpu.VMEM((_C,), jnp.float32)] * 2
        + [pltpu.VMEM((_C,), jnp.int32)] * 2
        + [pltpu.SemaphoreType.DMA] * 4
    ),
)(_pass_b_body)


def kernel(pos_xyz, cel_mat, sft_cel, spc, adj_ij, sft_idx, rc):
    cel = cel_mat[0]
    pos = pos_xyz[0] + spc[0].astype(jnp.float32) @ cel
    sftm = sft_cel @ cel
    sft_pad = jnp.pad(sftm, ((0, 5), (0, 0)))
    rc_f = jnp.asarray(rc, jnp.float32)
    rc2v = jnp.full((16,), rc_f * rc_f, jnp.float32)

    tabx = jnp.copy(pos[:, 0])
    taby = jnp.copy(pos[:, 1])
    tabz = jnp.copy(pos[:, 2])
    sftx = jnp.copy(sft_pad[:, 0])
    sfty = jnp.copy(sft_pad[:, 1])
    sftz = jnp.copy(sft_pad[:, 2])

    adj_flat = adj_ij.reshape(2 * _E)
    part = _pass_a(adj_flat, sft_idx, tabx, taby, sftx, sfty)
    adj_out_flat, sod = _pass_b(adj_flat, sft_idx, part, tabz, sftz, rc2v)
    return adj_out_flat.reshape(2, _E), sod


# speedup vs baseline: 98.9719x; 1.0073x over previous
import functools

import jax
import jax.numpy as jnp
from jax import lax
from jax.experimental import pallas as pl
from jax.experimental.pallas import tpu as pltpu
from jax.experimental.pallas import tpu_sc as plsc

_N = 50000
_E = 3200000
_NW = 32
_EW = _E // _NW
_C = 2000
_NCHUNK = _EW // _C
_NPAIR = _NCHUNK // 2

_mesh = plsc.VectorSubcoreMesh(core_axis_name="c", subcore_axis_name="s")
_params = pltpu.CompilerParams(needs_layout_passes=False)


def _worker_id():
    return lax.axis_index("s") * 2 + lax.axis_index("c")


def _body(adj_hbm, sft_hbm, tabx_hbm, taby_hbm, tabz_hbm,
          sftx_hbm, sfty_hbm, sftz_hbm, rc2_hbm,
          adj_out_hbm, sod_hbm, part_hbm,
          tabA, tabB, sftx, sfty, sftz, rc2v,
          bi0, bj0, bs0, bp0, bi1, bj1, bs1, bp1,
          bd0, b00, b10, bd1, b01, b11,
          sin0, sin1, sout0, sout1):
    base_w = _worker_id() * _EW
    pltpu.sync_copy(tabx_hbm, tabA)
    pltpu.sync_copy(taby_hbm, tabB)
    pltpu.sync_copy(sftx_hbm, sftx)
    pltpu.sync_copy(sfty_hbm, sfty)
    pltpu.sync_copy(sftz_hbm, sftz)
    pltpu.sync_copy(rc2_hbm, rc2v)
    rc2 = rc2v[...]
    neg1 = jnp.full((16,), -1, jnp.int32)

    ins = ((bi0, bj0, bs0, bp0, sin0), (bi1, bj1, bs1, bp1, sin1))
    outsA = ((bp0, sout0), (bp1, sout1))
    outsB = ((bd0, b00, b10, sout0), (bd1, b01, b11, sout1))

    def issue_in_a(s, b):
        bi, bj, bs, _, sin = ins[s]
        pltpu.async_copy(adj_hbm.at[pl.ds(b, _C)], bi, sin)
        pltpu.async_copy(adj_hbm.at[pl.ds(_E + b, _C)], bj, sin)
        pltpu.async_copy(sft_hbm.at[pl.ds(b, _C)], bs, sin)

    def wait_in_a(s):
        bi, bj, bs, _, sin = ins[s]
        pltpu.make_async_copy(adj_hbm.at[pl.ds(0, _C)], bi, sin).wait()
        pltpu.make_async_copy(adj_hbm.at[pl.ds(0, _C)], bj, sin).wait()
        pltpu.make_async_copy(sft_hbm.at[pl.ds(0, _C)], bs, sin).wait()

    def issue_in_b(s, b):
        bi, bj, bs, bp, sin = ins[s]
        issue_in_a(s, b)
        pltpu.async_copy(part_hbm.at[pl.ds(b, _C)], bp, sin)

    def wait_in_b(s):
        bi, bj, bs, bp, sin = ins[s]
        wait_in_a(s)
        pltpu.make_async_copy(part_hbm.at[pl.ds(0, _C)], bp, sin).wait()

    def compute_a(s):
        bi, bj, bs, bp, _ = ins[s]

        @plsc.parallel_loop(0, _C, step=16, unroll=5)
        def _(o):
            iv = bi[pl.ds(o, 16)]
            jv = bj[pl.ds(o, 16)]
            sv = bs[pl.ds(o, 16)]
            xj = plsc.load_gather(tabA, [jv])
            xi = plsc.load_gather(tabA, [iv])
            yj = plsc.load_gather(tabB, [jv])
            yi = plsc.load_gather(tabB, [iv])
            tx = plsc.load_gather(sftx, [sv])
            ty = plsc.load_gather(sfty, [sv])
            dx = xj - xi + tx
            dy = yj - yi + ty
            bp[pl.ds(o, 16)] = dx * dx + dy * dy

    def issue_out_a(s, b):
        bp, sout = outsA[s]
        pltpu.async_copy(bp, part_hbm.at[pl.ds(b, _C)], sout)

    def wait_out_a(s):
        bp, sout = outsA[s]
        pltpu.make_async_copy(bp, part_hbm.at[pl.ds(0, _C)], sout).wait()

    issue_in_a(0, base_w)

    def pair_a(t, carry):
        b0 = base_w + (2 * t) * _C
        b2 = jnp.minimum(b0 + 2 * _C, _E - _C)
        issue_in_a(1, b0 + _C)
        wait_in_a(0)

        @pl.when(t > 0)
        def _():
            wait_out_a(0)

        compute_a(0)
        issue_out_a(0, b0)
        issue_in_a(0, b2)
        wait_in_a(1)

        @pl.when(t > 0)
        def _():
            wait_out_a(1)

        compute_a(1)
        issue_out_a(1, b0 + _C)
        return carry

    lax.fori_loop(0, _NPAIR, pair_a, 0)
    wait_in_a(0)
    wait_out_a(0)
    wait_out_a(1)

    pltpu.sync_copy(tabz_hbm, tabA)

    def compute_b(s):
        bi, bj, bs, bp, _ = ins[s]
        bd, b0r, b1r, _ = outsB[s]

        @plsc.parallel_loop(0, _C, step=16, unroll=5)
        def _(o):
            iv = bi[pl.ds(o, 16)]
            jv = bj[pl.ds(o, 16)]
            sv = bs[pl.ds(o, 16)]
            pv = bp[pl.ds(o, 16)]
            zj = plsc.load_gather(tabA, [jv])
            zi = plsc.load_gather(tabA, [iv])
            tz = plsc.load_gather(sftz, [sv])
            dz = zj - zi + tz
            sod = pv + dz * dz
            m = sod <= rc2
            bd[pl.ds(o, 16)] = sod
            b0r[pl.ds(o, 16)] = jnp.where(m, iv, neg1)
            b1r[pl.ds(o, 16)] = jnp.where(m, jv, neg1)

    def issue_out_b(s, b):
        bd, b0r, b1r, sout = outsB[s]
        pltpu.async_copy(bd, sod_hbm.at[pl.ds(b, _C)], sout)
        pltpu.async_copy(b0r, adj_out_hbm.at[pl.ds(b, _C)], sout)
        pltpu.async_copy(b1r, adj_out_hbm.at[pl.ds(_E + b, _C)], sout)

    def wait_out_b(s):
        bd, b0r, b1r, sout = outsB[s]
        pltpu.make_async_copy(bd, sod_hbm.at[pl.ds(0, _C)], sout).wait()
        pltpu.make_async_copy(b0r, adj_out_hbm.at[pl.ds(0, _C)], sout).wait()
        pltpu.make_async_copy(b1r, adj_out_hbm.at[pl.ds(0, _C)], sout).wait()

    issue_in_b(0, base_w)

    def pair_b(t, carry):
        b0 = base_w + (2 * t) * _C
        b2 = jnp.minimum(b0 + 2 * _C, _E - _C)
        issue_in_b(1, b0 + _C)
        wait_in_b(0)

        @pl.when(t > 0)
        def _():
            wait_out_b(0)

        compute_b(0)
        issue_out_b(0, b0)
        issue_in_b(0, b2)
        wait_in_b(1)

        @pl.when(t > 0)
        def _():
            wait_out_b(1)

        compute_b(1)
        issue_out_b(1, b0 + _C)
        return carry

    lax.fori_loop(0, _NPAIR, pair_b, 0)
    wait_in_b(0)
    wait_out_b(0)
    wait_out_b(1)


_run = functools.partial(
    pl.kernel,
    out_type=(jax.ShapeDtypeStruct((2 * _E,), jnp.int32),
              jax.ShapeDtypeStruct((_E,), jnp.float32),
              jax.ShapeDtypeStruct((_E,), jnp.float32)),
    mesh=_mesh,
    compiler_params=_params,
    scratch_types=(
        [pltpu.VMEM((_N,), jnp.float32)] * 2
        + [pltpu.VMEM((32,), jnp.float32)] * 3
        + [pltpu.VMEM((16,), jnp.float32)]
        + [pltpu.VMEM((_C,), jnp.int32)] * 3 + [pltpu.VMEM((_C,), jnp.float32)]
        + [pltpu.VMEM((_C,), jnp.int32)] * 3 + [pltpu.VMEM((_C,), jnp.float32)]
        + [pltpu.VMEM((_C,), jnp.float32)] + [pltpu.VMEM((_C,), jnp.int32)] * 2
        + [pltpu.VMEM((_C,), jnp.float32)] + [pltpu.VMEM((_C,), jnp.int32)] * 2
        + [pltpu.SemaphoreType.DMA] * 4
    ),
)(_body)


def kernel(pos_xyz, cel_mat, sft_cel, spc, adj_ij, sft_idx, rc):
    cel = cel_mat[0]
    pos = pos_xyz[0] + spc[0].astype(jnp.float32) @ cel
    sftm = sft_cel @ cel
    sft_pad = jnp.pad(sftm, ((0, 5), (0, 0)))
    rc_f = jnp.asarray(rc, jnp.float32)
    rc2v = jnp.full((16,), rc_f * rc_f, jnp.float32)

    tabx = jnp.copy(pos[:, 0])
    taby = jnp.copy(pos[:, 1])
    tabz = jnp.copy(pos[:, 2])
    sftx = jnp.copy(sft_pad[:, 0])
    sfty = jnp.copy(sft_pad[:, 1])
    sftz = jnp.copy(sft_pad[:, 2])

    adj_flat = adj_ij.reshape(2 * _E)
    adj_out_flat, sod, _unused_part = _run(
        adj_flat, sft_idx, tabx, taby, tabz, sftx, sfty, sftz, rc2v)
    return adj_out_flat.reshape(2, _E), sod


# speedup vs baseline: 357.4402x; 3.6115x over previous
import functools

import jax
import jax.numpy as jnp
from jax import lax
from jax.experimental import pallas as pl
from jax.experimental.pallas import tpu as pltpu
from jax.experimental.pallas import tpu_sc as plsc

_N = 50000
_E = 3200000
_NW = 32
_EW = _E // _NW
_C = 2000
_NCHUNK = _EW // _C
_NPAIR = _NCHUNK // 2

_mesh = plsc.VectorSubcoreMesh(core_axis_name="c", subcore_axis_name="s")
_params = pltpu.CompilerParams(needs_layout_passes=False)


def _worker_id():
    return lax.axis_index("s") * 2 + lax.axis_index("c")


def _body(adj_hbm, sft_hbm, tabx_hbm, taby_hbm, tabz_hbm,
          sftx_hbm, sfty_hbm, sftz_hbm, rc2_hbm,
          a0_hbm, a1_hbm, sod_hbm, part_hbm,
          tabA, tabB, sftx, sfty, sftz, rc2v,
          bi0, bj0, bs0, bp0, bi1, bj1, bs1, bp1,
          bd0, b00, b10, bd1, b01, b11,
          sin0, sin1, sout0, sout1):
    base_w = _worker_id() * _EW
    pltpu.sync_copy(tabx_hbm, tabA)
    pltpu.sync_copy(taby_hbm, tabB)
    pltpu.sync_copy(sftx_hbm, sftx)
    pltpu.sync_copy(sfty_hbm, sfty)
    pltpu.sync_copy(sftz_hbm, sftz)
    pltpu.sync_copy(rc2_hbm, rc2v)
    rc2 = rc2v[...]
    neg1 = jnp.full((16,), -1, jnp.int32)

    ins = ((bi0, bj0, bs0, bp0, sin0), (bi1, bj1, bs1, bp1, sin1))
    outsA = ((bp0, sout0), (bp1, sout1))
    outsB = ((bd0, b00, b10, sout0), (bd1, b01, b11, sout1))

    def issue_in_a(s, b):
        bi, bj, bs, _, sin = ins[s]
        pltpu.async_copy(adj_hbm.at[pl.ds(b, _C)], bi, sin)
        pltpu.async_copy(adj_hbm.at[pl.ds(_E + b, _C)], bj, sin)
        pltpu.async_copy(sft_hbm.at[pl.ds(b, _C)], bs, sin)

    def wait_in_a(s):
        bi, bj, bs, _, sin = ins[s]
        pltpu.make_async_copy(adj_hbm.at[pl.ds(0, _C)], bi, sin).wait()
        pltpu.make_async_copy(adj_hbm.at[pl.ds(0, _C)], bj, sin).wait()
        pltpu.make_async_copy(sft_hbm.at[pl.ds(0, _C)], bs, sin).wait()

    def issue_in_b(s, b):
        bi, bj, bs, bp, sin = ins[s]
        issue_in_a(s, b)
        pltpu.async_copy(part_hbm.at[pl.ds(b, _C)], bp, sin)

    def wait_in_b(s):
        bi, bj, bs, bp, sin = ins[s]
        wait_in_a(s)
        pltpu.make_async_copy(part_hbm.at[pl.ds(0, _C)], bp, sin).wait()

    def compute_a(s):
        bi, bj, bs, bp, _ = ins[s]

        @plsc.parallel_loop(0, _C, step=16, unroll=5)
        def _(o):
            iv = bi[pl.ds(o, 16)]
            jv = bj[pl.ds(o, 16)]
            sv = bs[pl.ds(o, 16)]
            xj = plsc.load_gather(tabA, [jv])
            xi = plsc.load_gather(tabA, [iv])
            yj = plsc.load_gather(tabB, [jv])
            yi = plsc.load_gather(tabB, [iv])
            tx = plsc.load_gather(sftx, [sv])
            ty = plsc.load_gather(sfty, [sv])
            dx = xj - xi + tx
            dy = yj - yi + ty
            bp[pl.ds(o, 16)] = dx * dx + dy * dy

    def issue_out_a(s, b):
        bp, sout = outsA[s]
        pltpu.async_copy(bp, part_hbm.at[pl.ds(b, _C)], sout)

    def wait_out_a(s):
        bp, sout = outsA[s]
        pltpu.make_async_copy(bp, part_hbm.at[pl.ds(0, _C)], sout).wait()

    issue_in_a(0, base_w)

    def pair_a(t, carry):
        b0 = base_w + (2 * t) * _C
        b2 = jnp.minimum(b0 + 2 * _C, _E - _C)
        issue_in_a(1, b0 + _C)
        wait_in_a(0)

        @pl.when(t > 0)
        def _():
            wait_out_a(0)

        compute_a(0)
        issue_out_a(0, b0)
        issue_in_a(0, b2)
        wait_in_a(1)

        @pl.when(t > 0)
        def _():
            wait_out_a(1)

        compute_a(1)
        issue_out_a(1, b0 + _C)
        return carry

    lax.fori_loop(0, _NPAIR, pair_a, 0)
    wait_in_a(0)
    wait_out_a(0)
    wait_out_a(1)

    pltpu.sync_copy(tabz_hbm, tabA)

    def compute_b(s):
        bi, bj, bs, bp, _ = ins[s]
        bd, b0r, b1r, _ = outsB[s]

        @plsc.parallel_loop(0, _C, step=16, unroll=5)
        def _(o):
            iv = bi[pl.ds(o, 16)]
            jv = bj[pl.ds(o, 16)]
            sv = bs[pl.ds(o, 16)]
            pv = bp[pl.ds(o, 16)]
            zj = plsc.load_gather(tabA, [jv])
            zi = plsc.load_gather(tabA, [iv])
            tz = plsc.load_gather(sftz, [sv])
            dz = zj - zi + tz
            sod = pv + dz * dz
            m = sod <= rc2
            bd[pl.ds(o, 16)] = sod
            b0r[pl.ds(o, 16)] = jnp.where(m, iv, neg1)
            b1r[pl.ds(o, 16)] = jnp.where(m, jv, neg1)

    def issue_out_b(s, b):
        bd, b0r, b1r, sout = outsB[s]
        pltpu.async_copy(bd, sod_hbm.at[pl.ds(b, _C)], sout)
        pltpu.async_copy(b0r, a0_hbm.at[pl.ds(b, _C)], sout)
        pltpu.async_copy(b1r, a1_hbm.at[pl.ds(b, _C)], sout)

    def wait_out_b(s):
        bd, b0r, b1r, sout = outsB[s]
        pltpu.make_async_copy(bd, sod_hbm.at[pl.ds(0, _C)], sout).wait()
        pltpu.make_async_copy(b0r, a0_hbm.at[pl.ds(0, _C)], sout).wait()
        pltpu.make_async_copy(b1r, a1_hbm.at[pl.ds(0, _C)], sout).wait()

    issue_in_b(0, base_w)

    def pair_b(t, carry):
        b0 = base_w + (2 * t) * _C
        b2 = jnp.minimum(b0 + 2 * _C, _E - _C)
        issue_in_b(1, b0 + _C)
        wait_in_b(0)

        @pl.when(t > 0)
        def _():
            wait_out_b(0)

        compute_b(0)
        issue_out_b(0, b0)
        issue_in_b(0, b2)
        wait_in_b(1)

        @pl.when(t > 0)
        def _():
            wait_out_b(1)

        compute_b(1)
        issue_out_b(1, b0 + _C)
        return carry

    lax.fori_loop(0, _NPAIR, pair_b, 0)
    wait_in_b(0)
    wait_out_b(0)
    wait_out_b(1)


_run = functools.partial(
    pl.kernel,
    out_type=(jax.ShapeDtypeStruct((_E,), jnp.int32),
              jax.ShapeDtypeStruct((_E,), jnp.int32),
              jax.ShapeDtypeStruct((_E,), jnp.float32),
              jax.ShapeDtypeStruct((_E,), jnp.float32)),
    mesh=_mesh,
    compiler_params=_params,
    scratch_types=(
        [pltpu.VMEM((_N,), jnp.float32)] * 2
        + [pltpu.VMEM((32,), jnp.float32)] * 3
        + [pltpu.VMEM((16,), jnp.float32)]
        + [pltpu.VMEM((_C,), jnp.int32)] * 3 + [pltpu.VMEM((_C,), jnp.float32)]
        + [pltpu.VMEM((_C,), jnp.int32)] * 3 + [pltpu.VMEM((_C,), jnp.float32)]
        + [pltpu.VMEM((_C,), jnp.float32)] + [pltpu.VMEM((_C,), jnp.int32)] * 2
        + [pltpu.VMEM((_C,), jnp.float32)] + [pltpu.VMEM((_C,), jnp.int32)] * 2
        + [pltpu.SemaphoreType.DMA] * 4
    ),
)(_body)


def kernel(pos_xyz, cel_mat, sft_cel, spc, adj_ij, sft_idx, rc):
    cel = cel_mat[0]
    pos = pos_xyz[0] + spc[0].astype(jnp.float32) @ cel
    sftm = sft_cel @ cel
    sft_pad = jnp.pad(sftm, ((0, 5), (0, 0)))
    rc_f = jnp.asarray(rc, jnp.float32)
    rc2v = jnp.full((16,), rc_f * rc_f, jnp.float32)

    tabx = jnp.copy(pos[:, 0])
    taby = jnp.copy(pos[:, 1])
    tabz = jnp.copy(pos[:, 2])
    sftx = jnp.copy(sft_pad[:, 0])
    sfty = jnp.copy(sft_pad[:, 1])
    sftz = jnp.copy(sft_pad[:, 2])

    adj_flat = adj_ij.reshape(2 * _E)
    a0, a1, sod, _unused_part = _run(
        adj_flat, sft_idx, tabx, taby, tabz, sftx, sfty, sftz, rc2v)
    return jnp.stack([a0, a1], axis=0), sod
